# baseline (speedup 1.0000x reference)
import functools

import jax
import jax.numpy as jnp
from jax import lax
from jax.experimental import pallas as pl
from jax.experimental.pallas import tpu as pltpu
from jax.experimental.pallas import tpu_sc as plsc

_SQRT_HALF = 0.7071067811865476
_NC, _NS = 2, 16
_NW = _NC * _NS


def _router_body(x_ref, rw_ref, idx_ref, wn_ref, cnt_ref, aux_ref, pos_ref,
                 items_ref, *, blk, g2):
    x = x_ref[...]
    rw = rw_ref[...]
    n = x.shape[0]
    e = rw.shape[0]
    logits = jax.lax.dot_general(
        x, rw, (((1,), (1,)), ((), ())), preferred_element_type=jnp.float32)
    m = jnp.max(logits, axis=-1, keepdims=True)
    p = jnp.exp(logits - m)
    probs = p / jnp.sum(p, axis=-1, keepdims=True)
    iota = jax.lax.broadcasted_iota(jnp.int32, (n, e), 1)
    m0 = jnp.max(probs, axis=-1, keepdims=True)
    i0 = jnp.min(jnp.where(probs >= m0, iota, e), axis=-1, keepdims=True)
    probs1 = jnp.where(iota == i0, -1.0, probs)
    m1 = jnp.max(probs1, axis=-1, keepdims=True)
    i1 = jnp.min(jnp.where(probs1 >= m1, iota, e), axis=-1, keepdims=True)
    s01 = m0 + m1
    idx_ref[...] = jnp.concatenate([i0, i1], axis=1)
    wn_ref[...] = jnp.concatenate([m0 / s01, m1 / s01], axis=1)
    one_hot = (jnp.where(iota == i0, 1.0, 0.0)
               + jnp.where(iota == i1, 1.0, 0.0))
    cnt = jnp.sum(one_hot, axis=0, keepdims=True)
    cnt_ref[...] = cnt
    f = cnt / (n * 2.0)
    pbar = jnp.mean(probs, axis=0, keepdims=True)
    aux_ref[...] = jnp.sum(e * f * pbar, axis=-1, keepdims=True)
    ri = jax.lax.broadcasted_iota(jnp.int32, (n, n), 0)
    ci = jax.lax.broadcasted_iota(jnp.int32, (n, n), 1)
    tri = jnp.where(ci < ri, 1.0, 0.0)
    csum_excl = jax.lax.dot_general(tri, one_hot, (((1,), (0,)), ((), ())),
                                    precision=jax.lax.Precision.HIGHEST,
                                    preferred_element_type=jnp.float32)
    re = jax.lax.broadcasted_iota(jnp.int32, (e, e), 0)
    ce = jax.lax.broadcasted_iota(jnp.int32, (e, e), 1)
    m8 = jnp.where(re < ce, 1.0, 0.0)
    seg_start = jax.lax.dot_general(cnt, m8, (((1,), (0,)), ((), ())),
                                    precision=jax.lax.Precision.HIGHEST,
                                    preferred_element_type=jnp.float32)
    base = seg_start + csum_excl
    oh0 = jnp.where(iota == i0, 1.0, 0.0)
    p0v = jnp.sum(base * oh0, axis=-1, keepdims=True)
    oh1 = jnp.where(iota == i1, 1.0, 0.0)
    p1v = jnp.sum(base * oh1, axis=-1, keepdims=True)
    pos_ref[...] = jnp.concatenate([p0v, p1v], axis=1).astype(jnp.int32)
    hp = jax.lax.Precision.HIGHEST
    seg_end_f = seg_start + cnt
    nbv = (2 * n) // blk
    npairs = e * nbv
    piota = jax.lax.broadcasted_iota(jnp.int32, (1, npairs), 1)
    b_of_p = (piota % nbv).astype(jnp.float32)
    e_of_p = (piota // nbv).astype(jnp.float32)
    p_bc = jax.lax.broadcasted_iota(jnp.int32, (e, npairs), 1)
    e_bc = jax.lax.broadcasted_iota(jnp.int32, (e, npairs), 0)
    moh = jnp.where(p_bc // nbv == e_bc, 1.0, 0.0)
    dotp = lambda u, v: jax.lax.dot_general(
        u, v, (((1,), (0,)), ((), ())), precision=hp,
        preferred_element_type=jnp.float32)
    ss_p = dotp(seg_start, moh)
    se_p = dotp(seg_end_f, moh)
    ovp = (ss_p < (b_of_p + 1.0) * blk) & (se_p > b_of_p * blk)
    ovf = jnp.where(ovp, 1.0, 0.0)
    rp = jax.lax.broadcasted_iota(jnp.int32, (npairs, npairs), 0)
    cp = jax.lax.broadcasted_iota(jnp.int32, (npairs, npairs), 1)
    rank = dotp(ovf, jnp.where(rp < cp, 1.0, 0.0))
    ones_g = jnp.ones((1, g2), jnp.float32)
    outer = lambda u: jax.lax.dot_general(
        u, ones_g, (((0,), (0,)), ((), ())), precision=hp,
        preferred_element_type=jnp.float32)
    jio = jax.lax.broadcasted_iota(jnp.int32, (npairs, g2), 1)
    sel = jnp.where((outer(rank) == jio.astype(jnp.float32))
                    & (outer(ovf) > 0.5), 1.0, 0.0)
    n_act = jnp.sum(ovf, axis=-1, keepdims=True)
    lastoh = jnp.where(rank == n_act - 1.0, ovf, 0.0)
    jg = jax.lax.broadcasted_iota(jnp.int32, (1, g2), 1).astype(jnp.float32)
    inval = jnp.where(jg >= n_act, 1.0, 0.0)
    sel_al = sel + jax.lax.dot_general(
        lastoh, inval, (((0,), (0,)), ((), ())), precision=hp,
        preferred_element_type=jnp.float32)
    e_sel = dotp(e_of_p, sel_al)
    b_sel = dotp(b_of_p, sel_al)
    ss_sel = dotp(ss_p, sel)
    se_sel = dotp(se_p, sel)
    lo_sel = jnp.clip(ss_sel - b_sel * blk, 0.0, float(blk))
    hi_sel = jnp.clip(se_sel - b_sel * blk, 0.0, float(blk))
    items_ref[...] = jnp.concatenate(
        [e_sel, b_sel, lo_sel, hi_sel], axis=0).astype(jnp.int32)


def _gmm_body(ea_ref, rb_ref, lo_ref, hi_ref, xs_ref, up_ref, down_ref,
              out_ref, *, blk):
    f = pl.program_id(0)
    j = pl.program_id(1)

    @pl.when((f == 0) & (j == 0))
    def _():
        out_ref[...] = jnp.zeros_like(out_ref)

    lo = lo_ref[j]
    hi = hi_ref[j]

    @pl.when(hi > lo)
    def _():
        rb = rb_ref[j]
        xs = xs_ref[pl.ds(rb * blk, blk), :].astype(jnp.float32)
        h = jax.lax.dot_general(xs, up_ref[0], (((1,), (0,)), ((), ())),
                                preferred_element_type=jnp.float32)
        h = 0.5 * h * (1.0 + jax.lax.erf(h * _SQRT_HALF))
        riota = jax.lax.broadcasted_iota(jnp.int32, (blk, 1), 0)
        wm = jnp.where((riota >= lo) & (riota < hi), 1.0, 0.0)
        h = h * wm
        y = jax.lax.dot_general(h, down_ref[0], (((1,), (0,)), ((), ())),
                                preferred_element_type=jnp.float32)
        out_ref[pl.ds(rb * blk, blk), :] += y


def _sc_scatter_rows(flat, p0, p1, n_out):
    n, d = flat.shape
    per_w = n // _NW
    ch = min(16, per_w)
    mesh = plsc.VectorSubcoreMesh(core_axis_name="c", subcore_axis_name="s")

    @functools.partial(
        pl.kernel, mesh=mesh,
        out_type=jax.ShapeDtypeStruct((n_out, d), flat.dtype),
        scratch_types=[
            pltpu.VMEM((ch,), jnp.int32),
            pltpu.VMEM((ch, d), flat.dtype),
            pltpu.SemaphoreType.DMA,
        ])
    def k(flat_hbm, p0_hbm, p1_hbm, xs_hbm, idx_v, rows_v, sem):
        wid = lax.axis_index("s") * _NC + lax.axis_index("c")
        base = wid * per_w

        def body(i, carry):
            off = base + i * ch
            pltpu.sync_copy(flat_hbm.at[pl.ds(off, ch)], rows_v)
            pltpu.sync_copy(p0_hbm.at[pl.ds(off, ch)], idx_v)
            pltpu.async_copy(rows_v, xs_hbm.at[idx_v], sem).wait()
            pltpu.sync_copy(p1_hbm.at[pl.ds(off, ch)], idx_v)
            pltpu.async_copy(rows_v, xs_hbm.at[idx_v], sem).wait()
            return carry

        lax.fori_loop(0, per_w // ch, body, 0)

    return k(flat, p0, p1)


def _sc_gather_rows(table, idx):
    n_rows = idx.shape[0]
    d = table.shape[1]
    per_w = n_rows // _NW
    ch = min(32, per_w)
    mesh = plsc.VectorSubcoreMesh(core_axis_name="c", subcore_axis_name="s")

    @functools.partial(
        pl.kernel, mesh=mesh,
        out_type=jax.ShapeDtypeStruct((n_rows, d), table.dtype),
        scratch_types=[
            pltpu.VMEM((ch,), jnp.int32),
            pltpu.VMEM((ch, d), table.dtype),
            pltpu.SemaphoreType.DMA,
        ])
    def k(table_hbm, idx_hbm, out_hbm, idx_v, rows_v, sem):
        wid = lax.axis_index("s") * _NC + lax.axis_index("c")
        base = wid * per_w

        def body(i, carry):
            off = base + i * ch
            pltpu.sync_copy(idx_hbm.at[pl.ds(off, ch)], idx_v)
            pltpu.async_copy(table_hbm.at[idx_v], rows_v, sem).wait()
            pltpu.sync_copy(rows_v, out_hbm.at[pl.ds(off, ch)])
            return carry

        lax.fori_loop(0, per_w // ch, body, 0)

    return k(table, idx)


def _wadd_body(a_ref, b_ref, w_ref, o_ref):
    w = w_ref[...]
    o_ref[...] = a_ref[...] * w[:, 0:1] + b_ref[...] * w[:, 1:2]


def _sc_combine(y, p0, p1, wn):
    n = p0.shape[0]
    d = y.shape[1]
    gath = _sc_gather_rows(y, jnp.concatenate([p0, p1]))
    nb = n // 256
    return pl.pallas_call(
        _wadd_body,
        grid=(nb,),
        in_specs=[
            pl.BlockSpec((256, d), lambda i: (i, 0)),
            pl.BlockSpec((256, d), lambda i, _nb=nb: (i + _nb, 0)),
            pl.BlockSpec((256, 2), lambda i: (i, 0)),
        ],
        out_specs=pl.BlockSpec((256, d), lambda i: (i, 0)),
        out_shape=jax.ShapeDtypeStruct((n, d), y.dtype),
    )(gath, gath, wn)


def kernel(hidden_states, router_w, up_w, down_w):
    b, s, h_dim = hidden_states.shape
    n = b * s
    e = router_w.shape[0]
    ff = up_w.shape[2]
    flat = hidden_states.reshape(n, h_dim)

    a = n * 2
    blk = 256
    nb = a // blk
    g2 = nb + e - 1
    top_idx, top_wn, counts, aux, pos2, items = pl.pallas_call(
        functools.partial(_router_body, blk=blk, g2=g2),
        out_shape=(
            jax.ShapeDtypeStruct((n, 2), jnp.int32),
            jax.ShapeDtypeStruct((n, 2), jnp.float32),
            jax.ShapeDtypeStruct((1, e), jnp.float32),
            jax.ShapeDtypeStruct((1, 1), jnp.float32),
            jax.ShapeDtypeStruct((n, 2), jnp.int32),
            jax.ShapeDtypeStruct((4, g2), jnp.int32),
        ),
    )(flat, router_w)
    aux_loss = aux[0, 0]
    p0 = pos2[:, 0]
    p1 = pos2[:, 1]
    item_e = items[0]
    item_rb = items[1]
    lo_rel = items[2]
    hi_rel = items[3]

    xs = _sc_scatter_rows(flat, p0, p1, a).astype(jnp.bfloat16)

    bf = min(256, ff)
    ff_t = ff // bf
    grid_spec = pltpu.PrefetchScalarGridSpec(
        num_scalar_prefetch=4,
        grid=(ff_t, g2),
        in_specs=[
            pl.BlockSpec((a, h_dim), lambda f, j, ea, rb, lo, hi: (0, 0)),
            pl.BlockSpec((1, h_dim, bf),
                         lambda f, j, ea, rb, lo, hi: (ea[j], 0, f)),
            pl.BlockSpec((1, bf, h_dim),
                         lambda f, j, ea, rb, lo, hi: (ea[j], f, 0)),
        ],
        out_specs=pl.BlockSpec((a, h_dim), lambda f, j, ea, rb, lo, hi: (0, 0)),
    )
    y = pl.pallas_call(
        functools.partial(_gmm_body, blk=blk),
        grid_spec=grid_spec,
        out_shape=jax.ShapeDtypeStruct((a, h_dim), jnp.float32),
        compiler_params=pltpu.CompilerParams(
            dimension_semantics=("arbitrary", "arbitrary"),
            vmem_limit_bytes=63 * 1024 * 1024,
        ),
    )(item_e, item_rb, lo_rel, hi_rel, xs, up_w, down_w)

    out = _sc_combine(y, p0, p1, top_wn)

    return out.reshape(b, s, h_dim), aux_loss


# speedup vs baseline: 1.5336x; 1.5336x over previous
import functools

import jax
import jax.numpy as jnp
from jax import lax
from jax.experimental import pallas as pl
from jax.experimental.pallas import tpu as pltpu
from jax.experimental.pallas import tpu_sc as plsc

_SQRT_HALF = 0.7071067811865476
_NC, _NS = 2, 16
_NW = _NC * _NS


def _router_body(x_ref, rw_ref, idx_ref, wn_ref, cnt_ref, aux_ref, pos_ref,
                 items_ref, *, blk, g2):
    x = x_ref[...]
    rw = rw_ref[...]
    n = x.shape[0]
    e = rw.shape[0]
    logits = jax.lax.dot_general(
        x, rw, (((1,), (1,)), ((), ())), preferred_element_type=jnp.float32)
    m = jnp.max(logits, axis=-1, keepdims=True)
    p = jnp.exp(logits - m)
    probs = p / jnp.sum(p, axis=-1, keepdims=True)
    iota = jax.lax.broadcasted_iota(jnp.int32, (n, e), 1)
    m0 = jnp.max(probs, axis=-1, keepdims=True)
    i0 = jnp.min(jnp.where(probs >= m0, iota, e), axis=-1, keepdims=True)
    probs1 = jnp.where(iota == i0, -1.0, probs)
    m1 = jnp.max(probs1, axis=-1, keepdims=True)
    i1 = jnp.min(jnp.where(probs1 >= m1, iota, e), axis=-1, keepdims=True)
    s01 = m0 + m1
    idx_ref[...] = jnp.concatenate([i0, i1], axis=1)
    wn_ref[...] = jnp.concatenate([m0 / s01, m1 / s01], axis=1)
    one_hot = (jnp.where(iota == i0, 1.0, 0.0)
               + jnp.where(iota == i1, 1.0, 0.0))
    cnt = jnp.sum(one_hot, axis=0, keepdims=True)
    cnt_ref[...] = cnt
    f = cnt / (n * 2.0)
    pbar = jnp.mean(probs, axis=0, keepdims=True)
    aux_ref[...] = jnp.sum(e * f * pbar, axis=-1, keepdims=True)
    ri = jax.lax.broadcasted_iota(jnp.int32, (n, n), 0)
    ci = jax.lax.broadcasted_iota(jnp.int32, (n, n), 1)
    tri = jnp.where(ci < ri, 1.0, 0.0)
    csum_excl = jax.lax.dot_general(tri, one_hot, (((1,), (0,)), ((), ())),
                                    precision=jax.lax.Precision.HIGHEST,
                                    preferred_element_type=jnp.float32)
    re = jax.lax.broadcasted_iota(jnp.int32, (e, e), 0)
    ce = jax.lax.broadcasted_iota(jnp.int32, (e, e), 1)
    m8 = jnp.where(re < ce, 1.0, 0.0)
    seg_start = jax.lax.dot_general(cnt, m8, (((1,), (0,)), ((), ())),
                                    precision=jax.lax.Precision.HIGHEST,
                                    preferred_element_type=jnp.float32)
    base = seg_start + csum_excl
    oh0 = jnp.where(iota == i0, 1.0, 0.0)
    p0v = jnp.sum(base * oh0, axis=-1, keepdims=True)
    oh1 = jnp.where(iota == i1, 1.0, 0.0)
    p1v = jnp.sum(base * oh1, axis=-1, keepdims=True)
    pos_ref[...] = jnp.concatenate([p0v, p1v], axis=1).astype(jnp.int32)
    hp = jax.lax.Precision.HIGHEST
    seg_end_f = seg_start + cnt
    nbv = (2 * n) // blk
    npairs = e * nbv
    piota = jax.lax.broadcasted_iota(jnp.int32, (1, npairs), 1)
    b_of_p = (piota % nbv).astype(jnp.float32)
    e_of_p = (piota // nbv).astype(jnp.float32)
    p_bc = jax.lax.broadcasted_iota(jnp.int32, (e, npairs), 1)
    e_bc = jax.lax.broadcasted_iota(jnp.int32, (e, npairs), 0)
    moh = jnp.where(p_bc // nbv == e_bc, 1.0, 0.0)
    dotp = lambda u, v: jax.lax.dot_general(
        u, v, (((1,), (0,)), ((), ())), precision=hp,
        preferred_element_type=jnp.float32)
    ss_p = dotp(seg_start, moh)
    se_p = dotp(seg_end_f, moh)
    ovp = (ss_p < (b_of_p + 1.0) * blk) & (se_p > b_of_p * blk)
    ovf = jnp.where(ovp, 1.0, 0.0)
    rp = jax.lax.broadcasted_iota(jnp.int32, (npairs, npairs), 0)
    cp = jax.lax.broadcasted_iota(jnp.int32, (npairs, npairs), 1)
    rank = dotp(ovf, jnp.where(rp < cp, 1.0, 0.0))
    ones_g = jnp.ones((1, g2), jnp.float32)
    outer = lambda u: jax.lax.dot_general(
        u, ones_g, (((0,), (0,)), ((), ())), precision=hp,
        preferred_element_type=jnp.float32)
    jio = jax.lax.broadcasted_iota(jnp.int32, (npairs, g2), 1)
    sel = jnp.where((outer(rank) == jio.astype(jnp.float32))
                    & (outer(ovf) > 0.5), 1.0, 0.0)
    n_act = jnp.sum(ovf, axis=-1, keepdims=True)
    lastoh = jnp.where(rank == n_act - 1.0, ovf, 0.0)
    jg = jax.lax.broadcasted_iota(jnp.int32, (1, g2), 1).astype(jnp.float32)
    inval = jnp.where(jg >= n_act, 1.0, 0.0)
    sel_al = sel + jax.lax.dot_general(
        lastoh, inval, (((0,), (0,)), ((), ())), precision=hp,
        preferred_element_type=jnp.float32)
    e_sel = dotp(e_of_p, sel_al)
    b_sel = dotp(b_of_p, sel_al)
    ss_sel = dotp(ss_p, sel)
    se_sel = dotp(se_p, sel)
    lo_sel = jnp.clip(ss_sel - b_sel * blk, 0.0, float(blk))
    hi_sel = jnp.clip(se_sel - b_sel * blk, 0.0, float(blk))
    items_ref[...] = jnp.concatenate(
        [e_sel, b_sel, lo_sel, hi_sel], axis=0).astype(jnp.int32)


def _gmm_body(ea_ref, rb_ref, lo_ref, hi_ref, xs_ref, up_ref, down_ref,
              out_ref, *, blk):
    f = pl.program_id(0)
    j = pl.program_id(1)

    @pl.when((f == 0) & (j == 0))
    def _():
        out_ref[...] = jnp.zeros_like(out_ref)

    lo = lo_ref[j]
    hi = hi_ref[j]

    @pl.when(hi > lo)
    def _():
        rb = rb_ref[j]
        xs = xs_ref[...]
        h = jax.lax.dot_general(xs, up_ref[0], (((1,), (0,)), ((), ())),
                                preferred_element_type=jnp.float32)
        h = 0.5 * h * (1.0 + jax.lax.erf(h * _SQRT_HALF))
        riota = jax.lax.broadcasted_iota(jnp.int32, (blk, 1), 0)
        wm = jnp.where((riota >= lo) & (riota < hi), 1.0, 0.0)
        h = h * wm
        y = jax.lax.dot_general(h, down_ref[0], (((1,), (0,)), ((), ())),
                                preferred_element_type=jnp.float32)
        out_ref[pl.ds(rb * blk, blk), :] += y


def _sc_scatter_rows(flat, p0, p1, n_out):
    n, d = flat.shape
    per_w = n // _NW
    ch = min(16, per_w)
    mesh = plsc.VectorSubcoreMesh(core_axis_name="c", subcore_axis_name="s")

    @functools.partial(
        pl.kernel, mesh=mesh,
        out_type=jax.ShapeDtypeStruct((n_out, d), flat.dtype),
        scratch_types=[
            pltpu.VMEM((ch,), jnp.int32),
            pltpu.VMEM((ch, d), flat.dtype),
            pltpu.SemaphoreType.DMA,
        ])
    def k(flat_hbm, p0_hbm, p1_hbm, xs_hbm, idx_v, rows_v, sem):
        wid = lax.axis_index("s") * _NC + lax.axis_index("c")
        base = wid * per_w

        def body(i, carry):
            off = base + i * ch
            pltpu.sync_copy(flat_hbm.at[pl.ds(off, ch)], rows_v)
            pltpu.sync_copy(p0_hbm.at[pl.ds(off, ch)], idx_v)
            pltpu.async_copy(rows_v, xs_hbm.at[idx_v], sem).wait()
            pltpu.sync_copy(p1_hbm.at[pl.ds(off, ch)], idx_v)
            pltpu.async_copy(rows_v, xs_hbm.at[idx_v], sem).wait()
            return carry

        lax.fori_loop(0, per_w // ch, body, 0)

    return k(flat, p0, p1)


def _sc_gather_rows(table, idx):
    n_rows = idx.shape[0]
    d = table.shape[1]
    per_w = n_rows // _NW
    ch = min(32, per_w)
    mesh = plsc.VectorSubcoreMesh(core_axis_name="c", subcore_axis_name="s")

    @functools.partial(
        pl.kernel, mesh=mesh,
        out_type=jax.ShapeDtypeStruct((n_rows, d), table.dtype),
        scratch_types=[
            pltpu.VMEM((ch,), jnp.int32),
            pltpu.VMEM((ch, d), table.dtype),
            pltpu.SemaphoreType.DMA,
        ])
    def k(table_hbm, idx_hbm, out_hbm, idx_v, rows_v, sem):
        wid = lax.axis_index("s") * _NC + lax.axis_index("c")
        base = wid * per_w

        def body(i, carry):
            off = base + i * ch
            pltpu.sync_copy(idx_hbm.at[pl.ds(off, ch)], idx_v)
            pltpu.async_copy(table_hbm.at[idx_v], rows_v, sem).wait()
            pltpu.sync_copy(rows_v, out_hbm.at[pl.ds(off, ch)])
            return carry

        lax.fori_loop(0, per_w // ch, body, 0)

    return k(table, idx)


def _wadd_body(a_ref, b_ref, w_ref, o_ref):
    w = w_ref[...]
    o_ref[...] = a_ref[...] * w[:, 0:1] + b_ref[...] * w[:, 1:2]


def _sc_combine(y, p0, p1, wn):
    n = p0.shape[0]
    d = y.shape[1]
    gath = _sc_gather_rows(y, jnp.concatenate([p0, p1]))
    nb = n // 256
    return pl.pallas_call(
        _wadd_body,
        grid=(nb,),
        in_specs=[
            pl.BlockSpec((256, d), lambda i: (i, 0)),
            pl.BlockSpec((256, d), lambda i, _nb=nb: (i + _nb, 0)),
            pl.BlockSpec((256, 2), lambda i: (i, 0)),
        ],
        out_specs=pl.BlockSpec((256, d), lambda i: (i, 0)),
        out_shape=jax.ShapeDtypeStruct((n, d), y.dtype),
    )(gath, gath, wn)


def kernel(hidden_states, router_w, up_w, down_w):
    b, s, h_dim = hidden_states.shape
    n = b * s
    e = router_w.shape[0]
    ff = up_w.shape[2]
    flat = hidden_states.reshape(n, h_dim)

    a = n * 2
    blk = 512
    nb = a // blk
    g2 = nb + e - 1
    top_idx, top_wn, counts, aux, pos2, items = pl.pallas_call(
        functools.partial(_router_body, blk=blk, g2=g2),
        out_shape=(
            jax.ShapeDtypeStruct((n, 2), jnp.int32),
            jax.ShapeDtypeStruct((n, 2), jnp.float32),
            jax.ShapeDtypeStruct((1, e), jnp.float32),
            jax.ShapeDtypeStruct((1, 1), jnp.float32),
            jax.ShapeDtypeStruct((n, 2), jnp.int32),
            jax.ShapeDtypeStruct((4, g2), jnp.int32),
        ),
    )(flat, router_w)
    aux_loss = aux[0, 0]
    p0 = pos2[:, 0]
    p1 = pos2[:, 1]
    item_e = items[0]
    item_rb = items[1]
    lo_rel = items[2]
    hi_rel = items[3]

    xs = _sc_scatter_rows(flat, p0, p1, a)

    bf = min(512, ff)
    ff_t = ff // bf
    grid_spec = pltpu.PrefetchScalarGridSpec(
        num_scalar_prefetch=4,
        grid=(ff_t, g2),
        in_specs=[
            pl.BlockSpec((blk, h_dim), lambda f, j, ea, rb, lo, hi: (rb[j], 0)),
            pl.BlockSpec((1, h_dim, bf),
                         lambda f, j, ea, rb, lo, hi: (ea[j], 0, f)),
            pl.BlockSpec((1, bf, h_dim),
                         lambda f, j, ea, rb, lo, hi: (ea[j], f, 0)),
        ],
        out_specs=pl.BlockSpec((a, h_dim), lambda f, j, ea, rb, lo, hi: (0, 0)),
    )
    y = pl.pallas_call(
        functools.partial(_gmm_body, blk=blk),
        grid_spec=grid_spec,
        out_shape=jax.ShapeDtypeStruct((a, h_dim), jnp.float32),
        compiler_params=pltpu.CompilerParams(
            dimension_semantics=("arbitrary", "arbitrary"),
            vmem_limit_bytes=63 * 1024 * 1024,
        ),
    )(item_e, item_rb, lo_rel, hi_rel, xs, up_w, down_w)

    out = _sc_combine(y, p0, p1, top_wn)

    return out.reshape(b, s, h_dim), aux_loss
